# baseline (speedup 1.0000x reference)
import functools

import jax
import jax.numpy as jnp
from jax import lax
from jax.experimental import pallas as pl
from jax.experimental.pallas import tpu as pltpu
from jax.experimental.pallas import tpu_sc as plsc

HASH_BITS = 24
NUM_Q = 16384
FEAT = 512
NUM_BYTES = 2 ** (HASH_BITS - 3)
NUM_WORDS = 2 ** (HASH_BITS - 5)

NUM_SPLITS = 2
Q_SPLIT = NUM_Q // NUM_SPLITS

TC_BLOCK = 2048
TC_GRID = Q_SPLIT // TC_BLOCK

RP_GRID = 4
RP_BYTES_BLK = NUM_BYTES // RP_GRID
RP_WORDS_BLK = NUM_WORDS // RP_GRID

NUM_CORES = 2
NUM_SUBCORES = 16
NUM_WORKERS = NUM_CORES * NUM_SUBCORES
ROWS_PER_WORKER = Q_SPLIT // NUM_WORKERS
GATHER_CHUNK = 128
NUM_CHUNKS = ROWS_PER_WORKER // GATHER_CHUNK
LANES = 16


def _repack_tc_body(bset_ref, words_ref):
    bblk = jnp.reshape(bset_ref[...], (RP_BYTES_BLK // 128, 128))
    words_ref[...] = jnp.reshape(pltpu.bitcast(bblk, jnp.int32),
                                 (RP_WORDS_BLK,))


def _repack(binary_set):
    return pl.pallas_call(
        _repack_tc_body,
        grid=(RP_GRID,),
        in_specs=[pl.BlockSpec((RP_BYTES_BLK,), lambda i: (i,))],
        out_specs=pl.BlockSpec((RP_WORDS_BLK,), lambda i: (i,)),
        out_shape=jax.ShapeDtypeStruct((NUM_WORDS,), jnp.int32),
    )(binary_set)


def _hash_tc_body(x_ref, rm_ref, packed_ref):
    prod = jnp.dot(x_ref[...], rm_ref[...],
                   preferred_element_type=jnp.float32)
    signs = (prod < 0.0).astype(jnp.bfloat16)
    col = lax.broadcasted_iota(jnp.int32, (1, HASH_BITS), 1)
    pow2 = lax.shift_left(jnp.int32(1), col).astype(jnp.bfloat16)
    idx_f = lax.dot_general(pow2, signs,
                            (((1,), (1,)), ((), ())),
                            preferred_element_type=jnp.float32)
    h = idx_f.astype(jnp.int32)
    b = lax.shift_right_logical(h, 3)
    widx = jnp.bitwise_or(
        lax.shift_left(lax.shift_right_logical(b, 9), 7),
        jnp.bitwise_and(b, 127))
    k = jnp.bitwise_and(lax.shift_right_logical(b, 7), 3)
    bitpos = jnp.bitwise_or(lax.shift_left(k, 3), jnp.bitwise_and(h, 7))
    packed = jnp.bitwise_or(lax.shift_left(widx, 5), bitpos)
    packed_ref[...] = jnp.reshape(packed, (1, 1, TC_BLOCK))


def _hash_half(x_half, rm):
    return pl.pallas_call(
        _hash_tc_body,
        grid=(TC_GRID,),
        in_specs=[
            pl.BlockSpec((TC_BLOCK, FEAT), lambda i: (i, 0)),
            pl.BlockSpec((FEAT, HASH_BITS), lambda i: (0, 0)),
        ],
        out_specs=pl.BlockSpec((1, 1, TC_BLOCK), lambda i: (i, 0, 0)),
        out_shape=jax.ShapeDtypeStruct((TC_GRID, 1, TC_BLOCK), jnp.int32),
    )(x_half, rm)


def _lookup_sc_body(packed_hbm, words_hbm, out_hbm,
                    packed_v, widx_v, words_v, out_v, sem):
    wid = lax.axis_index("s") * NUM_CORES + lax.axis_index("c")
    base = wid * ROWS_PER_WORKER
    row = base // TC_BLOCK
    col = base % TC_BLOCK
    pltpu.sync_copy(packed_hbm.at[row, 0, pl.ds(col, ROWS_PER_WORKER)],
                    packed_v)
    copies = []
    for j in range(NUM_CHUNKS):
        for i in range(j * GATHER_CHUNK // LANES,
                       (j + 1) * GATHER_CHUNK // LANES):
            sl = pl.ds(i * LANES, LANES)
            widx_v[sl] = lax.shift_right_logical(packed_v[sl], 5)
        sl = pl.ds(j * GATHER_CHUNK, GATHER_CHUNK)
        copies.append(pltpu.async_copy(words_hbm.at[widx_v.at[sl]],
                                       words_v.at[sl], sem))
    for c in copies:
        c.wait()
    for i in range(ROWS_PER_WORKER // LANES):
        sl = pl.ds(i * LANES, LANES)
        out_v[sl] = jnp.bitwise_and(
            lax.shift_right_logical(words_v[sl],
                                    jnp.bitwise_and(packed_v[sl], 31)), 1)
    pltpu.sync_copy(out_v, out_hbm.at[pl.ds(base, ROWS_PER_WORKER)])


@functools.cache
def _lookup_bits_kernel():
    return pl.kernel(
        _lookup_sc_body,
        out_type=jax.ShapeDtypeStruct((Q_SPLIT,), jnp.int32),
        mesh=plsc.VectorSubcoreMesh(core_axis_name="c", subcore_axis_name="s",
                                    num_cores=NUM_CORES,
                                    num_subcores=NUM_SUBCORES),
        scratch_types=[
            pltpu.VMEM((ROWS_PER_WORKER,), jnp.int32),
            pltpu.VMEM((ROWS_PER_WORKER,), jnp.int32),
            pltpu.VMEM((ROWS_PER_WORKER,), jnp.int32),
            pltpu.VMEM((ROWS_PER_WORKER,), jnp.int32),
            pltpu.SemaphoreType.DMA,
        ],
    )


def kernel(x, is_training, test_local_stats, random_matrix, binary_set):
    x = jnp.reshape(x, (x.shape[0], -1))
    rm = jax.lax.stop_gradient(random_matrix)
    words = _repack(binary_set)
    lookup = _lookup_bits_kernel()
    halves = []
    for s in range(NUM_SPLITS):
        packed = _hash_half(
            lax.slice_in_dim(x, s * Q_SPLIT, (s + 1) * Q_SPLIT, axis=0), rm)
        halves.append(lookup(packed, words))
    bits = jnp.concatenate(halves)
    return bits.astype(jnp.bool_)


# speedup vs baseline: 1.6447x; 1.6447x over previous
import functools

import jax
import jax.numpy as jnp
from jax import lax
from jax.experimental import pallas as pl
from jax.experimental.pallas import tpu as pltpu
from jax.experimental.pallas import tpu_sc as plsc

HASH_BITS = 24
NUM_Q = 16384
FEAT = 512
NUM_BYTES = 2 ** (HASH_BITS - 3)
NUM_WORDS = 2 ** (HASH_BITS - 5)

TC_BLOCK = 4096
TC_GRID = NUM_Q // TC_BLOCK
BYTES_BLK = NUM_BYTES // TC_GRID
WORDS_BLK = NUM_WORDS // TC_GRID

NUM_CORES = 2
NUM_SUBCORES = 16
NUM_WORKERS = NUM_CORES * NUM_SUBCORES
ROWS_PER_WORKER = NUM_Q // NUM_WORKERS
GATHER_CHUNK = 128
NUM_CHUNKS = ROWS_PER_WORKER // GATHER_CHUNK
LANES = 16


def _hash_tc_body(x_ref, rm_ref, bset_any, packed_ref, words_ref,
                  buf0, buf1, sem0, sem1):
    i = pl.program_id(0)
    bufs, sems = (buf0, buf1), (sem0, sem1)

    def _copy_in(step, slot):
        return pltpu.make_async_copy(
            bset_any.at[pl.ds(step * BYTES_BLK, BYTES_BLK)], bufs[slot], sems[slot])

    @pl.when(i == 0)
    def _prime():
        _copy_in(0, 0).start()

    @pl.when((i + 1 < TC_GRID) & (i % 2 == 1))
    def _prefetch0():
        _copy_in(i + 1, 0).start()

    @pl.when((i + 1 < TC_GRID) & (i % 2 == 0))
    def _prefetch1():
        _copy_in(i + 1, 1).start()

    @pl.when(i % 2 == 0)
    def _w0():
        _copy_in(i, 0).wait()
        bblk = jnp.reshape(buf0[...], (BYTES_BLK // 128, 128))
        words_ref[...] = jnp.reshape(pltpu.bitcast(bblk, jnp.int32),
                                     (WORDS_BLK,))

    @pl.when(i % 2 == 1)
    def _w1():
        _copy_in(i, 1).wait()
        bblk = jnp.reshape(buf1[...], (BYTES_BLK // 128, 128))
        words_ref[...] = jnp.reshape(pltpu.bitcast(bblk, jnp.int32),
                                     (WORDS_BLK,))

    prod = jnp.dot(x_ref[...], rm_ref[...],
                   preferred_element_type=jnp.float32)
    signs = (prod < 0.0).astype(jnp.bfloat16)
    col = lax.broadcasted_iota(jnp.int32, (1, HASH_BITS), 1)
    pow2 = lax.shift_left(jnp.int32(1), col).astype(jnp.bfloat16)
    idx_f = lax.dot_general(pow2, signs,
                            (((1,), (1,)), ((), ())),
                            preferred_element_type=jnp.float32)
    h = idx_f.astype(jnp.int32)
    b = lax.shift_right_logical(h, 3)
    widx = jnp.bitwise_or(
        lax.shift_left(lax.shift_right_logical(b, 9), 7),
        jnp.bitwise_and(b, 127))
    k = jnp.bitwise_and(lax.shift_right_logical(b, 7), 3)
    bitpos = jnp.bitwise_or(lax.shift_left(k, 3), jnp.bitwise_and(h, 7))
    packed = jnp.bitwise_or(lax.shift_left(widx, 5), bitpos)
    packed_ref[...] = jnp.reshape(packed, (1, 1, TC_BLOCK))


def _hash_and_repack(x, rm, binary_set):
    return pl.pallas_call(
        _hash_tc_body,
        grid=(TC_GRID,),
        in_specs=[
            pl.BlockSpec((TC_BLOCK, FEAT), lambda i: (i, 0)),
            pl.BlockSpec((FEAT, HASH_BITS), lambda i: (0, 0)),
            pl.BlockSpec(memory_space=pl.ANY),
        ],
        out_specs=[
            pl.BlockSpec((1, 1, TC_BLOCK), lambda i: (i, 0, 0)),
            pl.BlockSpec((WORDS_BLK,), lambda i: (i,)),
        ],
        out_shape=[
            jax.ShapeDtypeStruct((TC_GRID, 1, TC_BLOCK), jnp.int32),
            jax.ShapeDtypeStruct((NUM_WORDS,), jnp.int32),
        ],
        scratch_shapes=[
            pltpu.VMEM((BYTES_BLK,), jnp.uint8),
            pltpu.VMEM((BYTES_BLK,), jnp.uint8),
            pltpu.SemaphoreType.DMA,
            pltpu.SemaphoreType.DMA,
        ],
    )(x, rm, binary_set)


def _lookup_sc_body(packed_hbm, words_hbm, out_hbm,
                    packed_v, widx_v, words_v, out_v, sem):
    wid = lax.axis_index("s") * NUM_CORES + lax.axis_index("c")
    base = wid * ROWS_PER_WORKER
    row = base // TC_BLOCK
    col = base % TC_BLOCK
    pltpu.sync_copy(packed_hbm.at[row, 0, pl.ds(col, ROWS_PER_WORKER)],
                    packed_v)
    copies = []
    for j in range(NUM_CHUNKS):
        for i in range(j * GATHER_CHUNK // LANES,
                       (j + 1) * GATHER_CHUNK // LANES):
            sl = pl.ds(i * LANES, LANES)
            widx_v[sl] = lax.shift_right_logical(packed_v[sl], 5)
        sl = pl.ds(j * GATHER_CHUNK, GATHER_CHUNK)
        copies.append(pltpu.async_copy(words_hbm.at[widx_v.at[sl]],
                                       words_v.at[sl], sem))
    for c in copies:
        c.wait()
    for i in range(ROWS_PER_WORKER // LANES):
        sl = pl.ds(i * LANES, LANES)
        out_v[sl] = jnp.bitwise_and(
            lax.shift_right_logical(words_v[sl],
                                    jnp.bitwise_and(packed_v[sl], 31)), 1)
    pltpu.sync_copy(out_v, out_hbm.at[pl.ds(base, ROWS_PER_WORKER)])


@functools.cache
def _lookup_bits_kernel():
    return pl.kernel(
        _lookup_sc_body,
        out_type=jax.ShapeDtypeStruct((NUM_Q,), jnp.int32),
        mesh=plsc.VectorSubcoreMesh(core_axis_name="c", subcore_axis_name="s",
                                    num_cores=NUM_CORES,
                                    num_subcores=NUM_SUBCORES),
        scratch_types=[
            pltpu.VMEM((ROWS_PER_WORKER,), jnp.int32),
            pltpu.VMEM((ROWS_PER_WORKER,), jnp.int32),
            pltpu.VMEM((ROWS_PER_WORKER,), jnp.int32),
            pltpu.VMEM((ROWS_PER_WORKER,), jnp.int32),
            pltpu.SemaphoreType.DMA,
        ],
    )


def kernel(x, is_training, test_local_stats, random_matrix, binary_set):
    x = jnp.reshape(x, (x.shape[0], -1))
    rm = jax.lax.stop_gradient(random_matrix)
    packed, words = _hash_and_repack(x, rm, binary_set)
    bits = _lookup_bits_kernel()(packed, words)
    return bits.astype(jnp.bool_)


# speedup vs baseline: 1.7426x; 1.0595x over previous
import functools

import jax
import jax.numpy as jnp
from jax import lax
from jax.experimental import pallas as pl
from jax.experimental.pallas import tpu as pltpu
from jax.experimental.pallas import tpu_sc as plsc

HASH_BITS = 24
NUM_Q = 16384
FEAT = 512
NUM_BYTES = 2 ** (HASH_BITS - 3)
NUM_WORDS = 2 ** (HASH_BITS - 5)

TC_BLOCK = 4096
TC_GRID = NUM_Q // TC_BLOCK
BYTES_BLK = NUM_BYTES // TC_GRID
WORDS_BLK = NUM_WORDS // TC_GRID

NUM_CORES = 2
NUM_SUBCORES = 16
NUM_WORKERS = NUM_CORES * NUM_SUBCORES
ROWS_PER_WORKER = NUM_Q // NUM_WORKERS
GATHER_CHUNK = 128
NUM_CHUNKS = ROWS_PER_WORKER // GATHER_CHUNK
LANES = 16


def _hash_tc_body(x_ref, rm_ref, bset_any, packed_ref, words_ref,
                  buf0, buf1, sem0, sem1):
    i = pl.program_id(0)
    bufs, sems = (buf0, buf1), (sem0, sem1)

    def _copy_in(step, slot):
        return pltpu.make_async_copy(
            bset_any.at[pl.ds(step * BYTES_BLK, BYTES_BLK)], bufs[slot], sems[slot])

    @pl.when(i == 0)
    def _prime():
        _copy_in(0, 0).start()

    @pl.when((i + 1 < TC_GRID) & (i % 2 == 1))
    def _prefetch0():
        _copy_in(i + 1, 0).start()

    @pl.when((i + 1 < TC_GRID) & (i % 2 == 0))
    def _prefetch1():
        _copy_in(i + 1, 1).start()

    @pl.when(i % 2 == 0)
    def _w0():
        _copy_in(i, 0).wait()
        bblk = jnp.reshape(buf0[...], (BYTES_BLK // 128, 128))
        words_ref[...] = jnp.reshape(pltpu.bitcast(bblk, jnp.int32),
                                     (WORDS_BLK,))

    @pl.when(i % 2 == 1)
    def _w1():
        _copy_in(i, 1).wait()
        bblk = jnp.reshape(buf1[...], (BYTES_BLK // 128, 128))
        words_ref[...] = jnp.reshape(pltpu.bitcast(bblk, jnp.int32),
                                     (WORDS_BLK,))

    prod_t = lax.dot_general(rm_ref[...], x_ref[...],
                             (((1,), (1,)), ((), ())),
                             preferred_element_type=jnp.float32
                             )
    signs = (prod_t < 0.0).astype(jnp.bfloat16)
    col = lax.broadcasted_iota(jnp.int32, (1, HASH_BITS), 1)
    pow2 = lax.shift_left(jnp.int32(1), col).astype(jnp.bfloat16)
    idx_f = lax.dot_general(pow2, signs,
                            (((1,), (0,)), ((), ())),
                            preferred_element_type=jnp.float32)
    h = idx_f.astype(jnp.int32)
    b = lax.shift_right_logical(h, 3)
    widx = jnp.bitwise_or(
        lax.shift_left(lax.shift_right_logical(b, 9), 7),
        jnp.bitwise_and(b, 127))
    k = jnp.bitwise_and(lax.shift_right_logical(b, 7), 3)
    bitpos = jnp.bitwise_or(lax.shift_left(k, 3), jnp.bitwise_and(h, 7))
    packed = jnp.bitwise_or(lax.shift_left(widx, 5), bitpos)
    packed_ref[...] = jnp.reshape(packed, (1, 1, TC_BLOCK))


def _hash_and_repack(x, rm, binary_set):
    return pl.pallas_call(
        _hash_tc_body,
        grid=(TC_GRID,),
        in_specs=[
            pl.BlockSpec((TC_BLOCK, FEAT), lambda i: (i, 0)),
            pl.BlockSpec((HASH_BITS, FEAT), lambda i: (0, 0)),
            pl.BlockSpec(memory_space=pl.ANY),
        ],
        out_specs=[
            pl.BlockSpec((1, 1, TC_BLOCK), lambda i: (i, 0, 0)),
            pl.BlockSpec((WORDS_BLK,), lambda i: (i,)),
        ],
        out_shape=[
            jax.ShapeDtypeStruct((TC_GRID, 1, TC_BLOCK), jnp.int32),
            jax.ShapeDtypeStruct((NUM_WORDS,), jnp.int32),
        ],
        scratch_shapes=[
            pltpu.VMEM((BYTES_BLK,), jnp.uint8),
            pltpu.VMEM((BYTES_BLK,), jnp.uint8),
            pltpu.SemaphoreType.DMA,
            pltpu.SemaphoreType.DMA,
        ],
    )(x, rm, binary_set)


def _lookup_sc_body(packed_hbm, words_hbm, out_hbm,
                    packed_v, widx_v, words_v, out_v, sem):
    wid = lax.axis_index("s") * NUM_CORES + lax.axis_index("c")
    base = wid * ROWS_PER_WORKER
    row = base // TC_BLOCK
    col = base % TC_BLOCK
    pltpu.sync_copy(packed_hbm.at[row, 0, pl.ds(col, ROWS_PER_WORKER)],
                    packed_v)
    copies = []
    for j in range(NUM_CHUNKS):
        for i in range(j * GATHER_CHUNK // LANES,
                       (j + 1) * GATHER_CHUNK // LANES):
            sl = pl.ds(i * LANES, LANES)
            widx_v[sl] = lax.shift_right_logical(packed_v[sl], 5)
        sl = pl.ds(j * GATHER_CHUNK, GATHER_CHUNK)
        copies.append(pltpu.async_copy(words_hbm.at[widx_v.at[sl]],
                                       words_v.at[sl], sem))
    for c in copies:
        c.wait()
    for i in range(ROWS_PER_WORKER // LANES):
        sl = pl.ds(i * LANES, LANES)
        out_v[sl] = jnp.bitwise_and(
            lax.shift_right_logical(words_v[sl],
                                    jnp.bitwise_and(packed_v[sl], 31)), 1)
    pltpu.sync_copy(out_v, out_hbm.at[pl.ds(base, ROWS_PER_WORKER)])


@functools.cache
def _lookup_bits_kernel():
    return pl.kernel(
        _lookup_sc_body,
        out_type=jax.ShapeDtypeStruct((NUM_Q,), jnp.int32),
        mesh=plsc.VectorSubcoreMesh(core_axis_name="c", subcore_axis_name="s",
                                    num_cores=NUM_CORES,
                                    num_subcores=NUM_SUBCORES),
        scratch_types=[
            pltpu.VMEM((ROWS_PER_WORKER,), jnp.int32),
            pltpu.VMEM((ROWS_PER_WORKER,), jnp.int32),
            pltpu.VMEM((ROWS_PER_WORKER,), jnp.int32),
            pltpu.VMEM((ROWS_PER_WORKER,), jnp.int32),
            pltpu.SemaphoreType.DMA,
        ],
    )


def kernel(x, is_training, test_local_stats, random_matrix, binary_set):
    x = jnp.reshape(x, (x.shape[0], -1))
    rm_t = jnp.transpose(jax.lax.stop_gradient(random_matrix))
    packed, words = _hash_and_repack(x, rm_t, binary_set)
    bits = _lookup_bits_kernel()(packed, words)
    return bits.astype(jnp.bool_)
